# Initial kernel scaffold; baseline (speedup 1.0000x reference)
import jax
import jax.numpy as jnp
from jax.experimental import pallas as pl


def kernel(embs, lengths, W_map, b_map, beg_seq_param, W_enc, b_enc):
    raise NotImplementedError("write your pallas kernel here")


# speedup vs baseline: 3.1524x; 3.1524x over previous
import functools

import jax
import jax.numpy as jnp
from jax import lax
from jax.experimental import pallas as pl
from jax.experimental.pallas import tpu as pltpu
from jax.experimental.pallas import tpu_sc as plsc

N_ROWS = 16384
H_IN = 512
B = 16
CH = 64
HALF = H_IN // 2
NVEC = HALF // 16


def _seg_sum_body(embs_hbm, len_hbm, out_hbm, len_v, buf, accv):
    b = lax.axis_index("s")
    h = lax.axis_index("c")
    col0 = h * HALF

    pltpu.sync_copy(len_hbm, len_v.at[pl.ds(0, B)])

    def cum_body(i, carry):
        cum, sb, eb = carry
        li = len_v[pl.ds(i, 16)][0]
        nc = cum + li
        sb = jnp.where(i == b, cum, sb)
        eb = jnp.where(i == b, nc, eb)
        return (nc, sb, eb)

    z = jnp.int32(0)
    _, s_b, e_b = lax.fori_loop(0, B, cum_body, (z, z, z))
    g0 = s_b & ~7
    nch = (e_b - g0 + CH - 1) >> 6

    def chunk_body(i, accs):
        off = g0 + i * CH
        cl = pl.multiple_of(jnp.minimum(off, N_ROWS - CH), 8)
        r_lo = jnp.maximum(s_b, off) - cl
        r_hi = jnp.minimum(e_b, off + CH) - cl
        pltpu.sync_copy(embs_hbm.at[pl.ds(cl, CH), pl.ds(col0, HALF)], buf)

        def row_body(r, a):
            return tuple(a[c] + buf[r, pl.ds(c * 16, 16)] for c in range(NVEC))

        return lax.fori_loop(r_lo, r_hi, row_body, accs)

    zero = jnp.zeros((16,), jnp.float32)
    accs = lax.fori_loop(0, nch, chunk_body, (zero,) * NVEC)
    for c in range(NVEC):
        accv[pl.ds(c * 16, 16)] = accs[c]
    pltpu.sync_copy(accv, out_hbm.at[pl.ds(b * H_IN + col0, HALF)])


def _make_seg_sum():
    mesh = plsc.VectorSubcoreMesh(core_axis_name="c", subcore_axis_name="s")
    return pl.kernel(
        _seg_sum_body,
        out_type=jax.ShapeDtypeStruct((B * H_IN,), jnp.float32),
        mesh=mesh,
        scratch_types=[
            pltpu.VMEM((2 * B,), jnp.int32),
            pltpu.VMEM((CH, HALF), jnp.float32),
            pltpu.VMEM((HALF,), jnp.float32),
        ],
    )


def _tail_body(seg_ref, lenf_ref, wm_ref, bm_ref, bs_ref, we_ref, be_ref, out_ref):
    seg = seg_ref[...]
    lenf = lenf_ref[...]
    summed = lax.dot_general(
        seg, wm_ref[...], (((1,), (1,)), ((), ())),
        preferred_element_type=jnp.float32,
    )
    summed = summed + lenf * bm_ref[...] + bs_ref[...]
    mean = summed / (lenf + 1.0)
    out = lax.dot_general(
        mean, we_ref[...], (((1,), (1,)), ((), ())),
        preferred_element_type=jnp.float32,
    )
    out_ref[...] = jnp.tanh(out + be_ref[...])


def kernel(embs, lengths, W_map, b_map, beg_seq_param, W_enc, b_enc):
    lengths = lengths.astype(jnp.int32)
    seg = _make_seg_sum()(embs, lengths).reshape(B, H_IN)
    lenf = lengths.astype(jnp.float32).reshape(B, 1)
    h_out = W_map.shape[0]
    out = pl.pallas_call(
        _tail_body,
        out_shape=jax.ShapeDtypeStruct((B, h_out), jnp.float32),
    )(seg, lenf, W_map, b_map.reshape(1, h_out),
      beg_seq_param.reshape(1, h_out), W_enc, b_enc.reshape(1, h_out))
    return out


# speedup vs baseline: 3.7376x; 1.1856x over previous
import functools

import jax
import jax.numpy as jnp
from jax import lax
from jax.experimental import pallas as pl
from jax.experimental.pallas import tpu as pltpu
from jax.experimental.pallas import tpu_sc as plsc

N_ROWS = 16384
H_IN = 512
B = 16
CH = 64
HALF = H_IN // 2
NVEC = HALF // 16


def _seg_sum_body(embs_hbm, len_hbm, out_hbm, len_v, buf, accv, sem0, sem1):
    b = lax.axis_index("s")
    h = lax.axis_index("c")
    col0 = h * HALF

    pltpu.sync_copy(len_hbm, len_v.at[pl.ds(0, B)])

    def cum_body(i, carry):
        cum, sb, eb = carry
        li = len_v[pl.ds(i, 16)][0]
        nc = cum + li
        sb = jnp.where(i == b, cum, sb)
        eb = jnp.where(i == b, nc, eb)
        return (nc, sb, eb)

    z = jnp.int32(0)
    _, s_b, e_b = lax.fori_loop(0, B, cum_body, (z, z, z))
    g0 = s_b & ~7
    nch = (e_b - g0 + CH - 1) >> 6
    sems = (sem0, sem1)

    def chunk_src(i):
        off = g0 + i * CH
        cl = pl.multiple_of(jnp.minimum(off, N_ROWS - CH), 8)
        return embs_hbm.at[pl.ds(cl, CH), pl.ds(col0, HALF)]

    def start(i, slot):
        pltpu.async_copy(chunk_src(i), buf.at[slot], sems[slot])

    def wait(slot):
        pltpu.make_async_copy(
            embs_hbm.at[pl.ds(0, CH), pl.ds(col0, HALF)], buf.at[slot], sems[slot]
        ).wait()

    def accum(i, slot, accs):
        off = g0 + i * CH
        cl = jnp.minimum(off, N_ROWS - CH)
        active = i < nch
        r_lo = jnp.where(active, jnp.maximum(s_b, off) - cl, z)
        r_hi = jnp.where(active, jnp.minimum(e_b, off + CH) - cl, z)

        def row_body(r, a):
            return tuple(
                a[c] + buf[slot, r, pl.ds(c * 16, 16)] for c in range(NVEC)
            )

        return lax.fori_loop(r_lo, r_hi, row_body, accs)

    @pl.when(nch > 0)
    def _():
        start(0, 0)

    def pair_body(p, accs):
        i0 = 2 * p
        i1 = i0 + 1
        wait(0)

        @pl.when(i1 < nch)
        def _():
            start(i1, 1)

        accs = accum(i0, 0, accs)

        @pl.when(i1 < nch)
        def _():
            wait(1)

        @pl.when(i1 + 1 < nch)
        def _():
            start(i1 + 1, 0)

        return accum(i1, 1, accs)

    zero = jnp.zeros((16,), jnp.float32)
    npairs = (nch + 1) >> 1
    accs = lax.fori_loop(0, npairs, pair_body, (zero,) * NVEC)
    for c in range(NVEC):
        accv[pl.ds(c * 16, 16)] = accs[c]
    pltpu.sync_copy(accv, out_hbm.at[pl.ds(b * H_IN + col0, HALF)])


def _make_seg_sum():
    mesh = plsc.VectorSubcoreMesh(core_axis_name="c", subcore_axis_name="s")
    return pl.kernel(
        _seg_sum_body,
        out_type=jax.ShapeDtypeStruct((B * H_IN,), jnp.float32),
        mesh=mesh,
        scratch_types=[
            pltpu.VMEM((2 * B,), jnp.int32),
            pltpu.VMEM((2, CH, HALF), jnp.float32),
            pltpu.VMEM((HALF,), jnp.float32),
            pltpu.SemaphoreType.DMA,
            pltpu.SemaphoreType.DMA,
        ],
    )


def _tail_body(seg_ref, lenf_ref, wm_ref, bm_ref, bs_ref, we_ref, be_ref, out_ref):
    seg = seg_ref[...]
    lenf = lenf_ref[...]
    summed = lax.dot_general(
        seg, wm_ref[...], (((1,), (1,)), ((), ())),
        preferred_element_type=jnp.float32,
    )
    summed = summed + lenf * bm_ref[...] + bs_ref[...]
    mean = summed / (lenf + 1.0)
    out = lax.dot_general(
        mean, we_ref[...], (((1,), (1,)), ((), ())),
        preferred_element_type=jnp.float32,
    )
    out_ref[...] = jnp.tanh(out + be_ref[...])


def kernel(embs, lengths, W_map, b_map, beg_seq_param, W_enc, b_enc):
    lengths = lengths.astype(jnp.int32)
    seg = _make_seg_sum()(embs, lengths).reshape(B, H_IN)
    lenf = lengths.astype(jnp.float32).reshape(B, 1)
    h_out = W_map.shape[0]
    out = pl.pallas_call(
        _tail_body,
        out_shape=jax.ShapeDtypeStruct((B, h_out), jnp.float32),
    )(seg, lenf, W_map, b_map.reshape(1, h_out),
      beg_seq_param.reshape(1, h_out), W_enc, b_enc.reshape(1, h_out))
    return out


# speedup vs baseline: 4.1320x; 1.1055x over previous
import functools

import jax
import jax.numpy as jnp
from jax import lax
from jax.experimental import pallas as pl
from jax.experimental.pallas import tpu as pltpu
from jax.experimental.pallas import tpu_sc as plsc

N_ROWS = 16384
H_IN = 512
B = 16
CH = 128
CH_SHIFT = CH.bit_length() - 1
HALF = H_IN // 2
NVEC = HALF // 16


def _seg_sum_body(embs_hbm, len_hbm, out_hbm, len_v, buf, accv, sem0, sem1):
    b = lax.axis_index("s")
    h = lax.axis_index("c")
    col0 = h * HALF

    pltpu.sync_copy(len_hbm, len_v.at[pl.ds(0, B)])

    def cum_body(i, carry):
        cum, sb, eb = carry
        li = len_v[pl.ds(i, 16)][0]
        nc = cum + li
        sb = jnp.where(i == b, cum, sb)
        eb = jnp.where(i == b, nc, eb)
        return (nc, sb, eb)

    z = jnp.int32(0)
    _, s_b, e_b = lax.fori_loop(0, B, cum_body, (z, z, z))
    g0 = s_b & ~7
    nch = (e_b - g0 + CH - 1) >> CH_SHIFT
    sems = (sem0, sem1)

    def chunk_src(i):
        off = g0 + i * CH
        cl = pl.multiple_of(jnp.minimum(off, N_ROWS - CH), 8)
        return embs_hbm.at[pl.ds(cl, CH), pl.ds(col0, HALF)]

    def start(i, slot):
        pltpu.async_copy(chunk_src(i), buf.at[slot], sems[slot])

    def wait(slot):
        pltpu.make_async_copy(
            embs_hbm.at[pl.ds(0, CH), pl.ds(col0, HALF)], buf.at[slot], sems[slot]
        ).wait()

    def accum(i, slot, accs):
        off = g0 + i * CH
        cl = jnp.minimum(off, N_ROWS - CH)
        active = i < nch
        r_lo = jnp.where(active, jnp.maximum(s_b, off) - cl, z)
        r_hi = jnp.where(active, jnp.minimum(e_b, off + CH) - cl, z)

        def row_body(r, a):
            return tuple(
                a[c] + buf[slot, r, pl.ds(c * 16, 16)] for c in range(NVEC)
            )

        return lax.fori_loop(r_lo, r_hi, row_body, accs)

    @pl.when(nch > 0)
    def _():
        start(0, 0)

    def pair_body(p, accs):
        i0 = 2 * p
        i1 = i0 + 1
        wait(0)

        @pl.when(i1 < nch)
        def _():
            start(i1, 1)

        accs = accum(i0, 0, accs)

        @pl.when(i1 < nch)
        def _():
            wait(1)

        @pl.when(i1 + 1 < nch)
        def _():
            start(i1 + 1, 0)

        return accum(i1, 1, accs)

    zero = jnp.zeros((16,), jnp.float32)
    npairs = (nch + 1) >> 1
    accs = lax.fori_loop(0, npairs, pair_body, (zero,) * NVEC)
    for c in range(NVEC):
        accv[pl.ds(c * 16, 16)] = accs[c]
    pltpu.sync_copy(accv, out_hbm.at[pl.ds(b * H_IN + col0, HALF)])


def _make_seg_sum():
    mesh = plsc.VectorSubcoreMesh(core_axis_name="c", subcore_axis_name="s")
    return pl.kernel(
        _seg_sum_body,
        out_type=jax.ShapeDtypeStruct((B * H_IN,), jnp.float32),
        mesh=mesh,
        scratch_types=[
            pltpu.VMEM((2 * B,), jnp.int32),
            pltpu.VMEM((2, CH, HALF), jnp.float32),
            pltpu.VMEM((HALF,), jnp.float32),
            pltpu.SemaphoreType.DMA,
            pltpu.SemaphoreType.DMA,
        ],
    )


def _tail_body(seg_ref, lenf_ref, wm_ref, bm_ref, bs_ref, we_ref, be_ref, out_ref):
    seg = seg_ref[...]
    lenf = lenf_ref[...]
    summed = lax.dot_general(
        seg, wm_ref[...], (((1,), (1,)), ((), ())),
        preferred_element_type=jnp.float32,
    )
    summed = summed + lenf * bm_ref[...] + bs_ref[...]
    mean = summed / (lenf + 1.0)
    out = lax.dot_general(
        mean, we_ref[...], (((1,), (1,)), ((), ())),
        preferred_element_type=jnp.float32,
    )
    out_ref[...] = jnp.tanh(out + be_ref[...])


def kernel(embs, lengths, W_map, b_map, beg_seq_param, W_enc, b_enc):
    lengths = lengths.astype(jnp.int32)
    seg = _make_seg_sum()(embs, lengths).reshape(B, H_IN)
    lenf = lengths.astype(jnp.float32).reshape(B, 1)
    h_out = W_map.shape[0]
    out = pl.pallas_call(
        _tail_body,
        out_shape=jax.ShapeDtypeStruct((B, h_out), jnp.float32),
    )(seg, lenf, W_map, b_map.reshape(1, h_out),
      beg_seq_param.reshape(1, h_out), W_enc, b_enc.reshape(1, h_out))
    return out


# speedup vs baseline: 4.3109x; 1.0433x over previous
import functools

import jax
import jax.numpy as jnp
from jax import lax
from jax.experimental import pallas as pl
from jax.experimental.pallas import tpu as pltpu
from jax.experimental.pallas import tpu_sc as plsc

N_ROWS = 16384
H_IN = 512
B = 16
CH = 128
CH_SHIFT = CH.bit_length() - 1
HALF = H_IN // 2
NVEC = HALF // 16


def _seg_sum_body(embs_hbm, len_hbm, out_hbm, len_v, buf, accv, sem0, sem1):
    b = lax.axis_index("s")
    h = lax.axis_index("c")
    col0 = h * HALF

    pltpu.sync_copy(len_hbm, len_v.at[pl.ds(0, B)])

    def cum_body(i, carry):
        cum, sb, eb = carry
        li = len_v[pl.ds(i, 16)][0]
        nc = cum + li
        sb = jnp.where(i == b, cum, sb)
        eb = jnp.where(i == b, nc, eb)
        return (nc, sb, eb)

    z = jnp.int32(0)
    _, s_b, e_b = lax.fori_loop(0, B, cum_body, (z, z, z))
    g0 = s_b & ~7
    nch = (e_b - g0 + CH - 1) >> CH_SHIFT
    sems = (sem0, sem1)

    def chunk_src(i):
        off = g0 + i * CH
        cl = pl.multiple_of(jnp.minimum(off, N_ROWS - CH), 8)
        return embs_hbm.at[pl.ds(cl, CH), pl.ds(col0, HALF)]

    def start(i, slot):
        pltpu.async_copy(chunk_src(i), buf.at[slot], sems[slot])

    def wait(slot):
        pltpu.make_async_copy(
            embs_hbm.at[pl.ds(0, CH), pl.ds(col0, HALF)], buf.at[slot], sems[slot]
        ).wait()

    def accum(i, slot, accs):
        off = g0 + i * CH
        cl = jnp.minimum(off, N_ROWS - CH)
        active = i < nch
        r_lo = jnp.where(active, jnp.maximum(s_b, off) - cl, z)
        r_hi = jnp.where(active, jnp.minimum(e_b, off + CH) - cl, z)

        def row_body(r, a):
            return tuple(
                a[c] + buf[slot, r, pl.ds(c * 16, 16)] for c in range(NVEC)
            )

        return lax.fori_loop(r_lo, r_hi, row_body, accs)

    @pl.when(nch > 0)
    def _():
        start(0, 0)

    def pair_body(p, accs):
        i0 = 2 * p
        i1 = i0 + 1
        wait(0)

        @pl.when(i1 < nch)
        def _():
            start(i1, 1)

        accs = accum(i0, 0, accs)

        @pl.when(i1 < nch)
        def _():
            wait(1)

        @pl.when(i1 + 1 < nch)
        def _():
            start(i1 + 1, 0)

        return accum(i1, 1, accs)

    zero = jnp.zeros((16,), jnp.float32)
    npairs = (nch + 1) >> 1
    accs = lax.fori_loop(0, npairs, pair_body, (zero,) * NVEC)
    for c in range(NVEC):
        accv[pl.ds(c * 16, 16)] = accs[c]
    c0 = 2 * h
    pltpu.sync_copy(accv.at[pl.ds(0, 128)],
                    out_hbm.at[pl.ds((c0 * B + b) * 128, 128)])
    pltpu.sync_copy(accv.at[pl.ds(128, 128)],
                    out_hbm.at[pl.ds(((c0 + 1) * B + b) * 128, 128)])


def _make_seg_sum():
    mesh = plsc.VectorSubcoreMesh(core_axis_name="c", subcore_axis_name="s")
    return pl.kernel(
        _seg_sum_body,
        out_type=jax.ShapeDtypeStruct((B * H_IN,), jnp.float32),
        mesh=mesh,
        scratch_types=[
            pltpu.VMEM((2 * B,), jnp.int32),
            pltpu.VMEM((2, CH, HALF), jnp.float32),
            pltpu.VMEM((HALF,), jnp.float32),
            pltpu.SemaphoreType.DMA,
            pltpu.SemaphoreType.DMA,
        ],
    )


def _tail_body(seg_ref, lenf_ref, wm_ref, bm_ref, bs_ref, we_ref, be_ref, out_ref):
    lenf = lenf_ref[...]
    summed = lenf * bm_ref[...] + bs_ref[...]
    for c in range(4):
        summed = summed + lax.dot_general(
            seg_ref[pl.ds(c * B, B), :], wm_ref[:, pl.ds(c * 128, 128)],
            (((1,), (1,)), ((), ())),
            preferred_element_type=jnp.float32,
        )
    mean = summed / (lenf + 1.0)
    out = lax.dot_general(
        mean, we_ref[...], (((1,), (1,)), ((), ())),
        preferred_element_type=jnp.float32,
    )
    out_ref[...] = jnp.tanh(out + be_ref[...])


def kernel(embs, lengths, W_map, b_map, beg_seq_param, W_enc, b_enc):
    lengths = lengths.astype(jnp.int32)
    seg = _make_seg_sum()(embs, lengths).reshape(4 * B, 128)
    lenf = lengths.astype(jnp.float32).reshape(B, 1)
    h_out = W_map.shape[0]
    out = pl.pallas_call(
        _tail_body,
        out_shape=jax.ShapeDtypeStruct((B, h_out), jnp.float32),
    )(seg, lenf, W_map, b_map.reshape(1, h_out),
      beg_seq_param.reshape(1, h_out), W_enc, b_enc.reshape(1, h_out))
    return out
